# baseline (speedup 1.0000x reference)
import functools

import jax
import jax.numpy as jnp
from jax import lax
from jax.experimental import pallas as pl
from jax.experimental.pallas import tpu as pltpu
from jax.experimental.pallas import tpu_sc as plsc

N_NODES_ = 100000
N_TYPES_ = 64
D_ = 128
BLK_ = 5000

NW_ = 32
CHUNK_ = 400
NCHUNK_ = N_NODES_ // CHUNK_
KMAX_ = (NCHUNK_ + NW_ - 1) // NW_


def _tc_onehot_body(idx_ref, eye_ref, oh_ref):
    idx = idx_ref[0]
    iota = lax.broadcasted_iota(jnp.int32, (N_TYPES_, BLK_), 0)
    onehot_t = (idx == iota).astype(jnp.float32)
    oh_ref[...] = lax.dot_general(
        onehot_t, eye_ref[...], (((0,), (0,)), ((), ())),
        preferred_element_type=jnp.float32)


def _sc_gather_body(w_hbm, idx_hbm, out_hbm, idx_v, rows_v, sem):
    wid = lax.axis_index("s") * 2 + lax.axis_index("c")
    for k in range(KMAX_):
        c = wid + NW_ * k

        @pl.when(c < NCHUNK_)
        def _():
            base = c * CHUNK_
            pltpu.sync_copy(idx_hbm.at[pl.ds(base, CHUNK_)], idx_v)
            pltpu.async_copy(w_hbm.at[idx_v], rows_v, sem).wait()
            pltpu.sync_copy(rows_v, out_hbm.at[pl.ds(base, CHUNK_)])


@functools.partial(
    pl.kernel,
    mesh=plsc.VectorSubcoreMesh(core_axis_name="c", subcore_axis_name="s"),
    out_type=jax.ShapeDtypeStruct((N_NODES_, D_), jnp.float32),
    scratch_types=[
        pltpu.VMEM((CHUNK_,), jnp.int32),
        pltpu.VMEM((CHUNK_, D_), jnp.float32),
        pltpu.SemaphoreType.DMA,
    ],
)
def _sc_gather(w_hbm, idx_hbm, out_hbm, idx_v, rows_v, sem):
    _sc_gather_body(w_hbm, idx_hbm, out_hbm, idx_v, rows_v, sem)


def kernel(node_atom, W, b):
    idx = node_atom.astype(jnp.int32)
    table = W + b[None, :]
    emb = _sc_gather(table, idx)

    idx3 = idx.reshape(N_NODES_ // BLK_, 1, BLK_)
    eye = jnp.eye(N_TYPES_, dtype=jnp.float32)
    oh = pl.pallas_call(
        _tc_onehot_body,
        grid=(N_NODES_ // BLK_,),
        in_specs=[
            pl.BlockSpec((1, 1, BLK_), lambda i: (i, 0, 0)),
            pl.BlockSpec((N_TYPES_, N_TYPES_), lambda i: (0, 0)),
        ],
        out_specs=pl.BlockSpec((BLK_, N_TYPES_), lambda i: (i, 0)),
        out_shape=jax.ShapeDtypeStruct((N_NODES_, N_TYPES_), jnp.float32),
    )(idx3, eye)
    return (emb, oh, oh)
